# baseline (speedup 1.0000x reference)
import functools

import jax
import jax.numpy as jnp
from jax import lax
from jax.experimental import pallas as pl
from jax.experimental.pallas import tpu as pltpu
from jax.experimental.pallas import tpu_sc as plsc

_LANES = 16


@functools.lru_cache(maxsize=None)
def _build_gather(B, N, K, D, KP):
    info = plsc.get_sparse_core_info()
    NC, NS = info.num_cores, info.num_subcores
    NW = NC * NS
    assert B % NW == 0
    bpw = B // NW
    rows_pw = bpw * K
    CH = 128
    assert rows_pw % CH == 0
    nchunk = rows_pw // CH

    mesh = plsc.VectorSubcoreMesh(core_axis_name="c", subcore_axis_name="s")

    @functools.partial(
        pl.kernel,
        mesh=mesh,
        compiler_params=pltpu.CompilerParams(
            use_tc_tiling_on_sc=False, needs_layout_passes=False
        ),
        out_type=jax.ShapeDtypeStruct((B * K, D), jnp.float32),
        scratch_types=[
            pltpu.VMEM((KP,), jnp.int32),
            pltpu.VMEM((CH,), jnp.int32),
            pltpu.VMEM((CH, D), jnp.float32),
            pltpu.SemaphoreType.DMA,
        ],
    )
    def gather_kernel(table_hbm, idx_hbm, out_hbm, idx_v, cidx_v, rows_v, sem):
        wid = lax.axis_index("s") * NC + lax.axis_index("c")
        pltpu.sync_copy(idx_hbm, idx_v)
        row0 = wid * rows_pw
        batch0 = wid * bpw

        def chunk_body(c, carry):
            p0 = c * CH
            for t in range(CH // _LANES):
                p = p0 + t * _LANES + lax.iota(jnp.int32, _LANES)
                b = lax.div(p, K)
                j = p - b * K
                src = (batch0 + b) * N + plsc.load_gather(idx_v, [j])
                cidx_v[pl.ds(t * _LANES, _LANES)] = src
            pltpu.async_copy(table_hbm.at[cidx_v], rows_v, sem).wait()
            pltpu.sync_copy(rows_v, out_hbm.at[pl.ds(row0 + p0, CH)])
            return carry

        lax.fori_loop(0, nchunk, chunk_body, 0)

    return gather_kernel


def kernel(tensor, indices):
    B, N, D = tensor.shape
    K = indices.shape[0]
    KP = (K + _LANES - 1) // _LANES * _LANES
    table = tensor.reshape(B * N, D)
    idx32 = jnp.pad(indices.astype(jnp.int32), (0, KP - K))
    out = _build_gather(B, N, K, D, KP)(table, idx32)
    return out.reshape(B, K, D)


# speedup vs baseline: 1.0508x; 1.0508x over previous
import functools

import jax
import jax.numpy as jnp
from jax import lax
from jax.experimental import pallas as pl
from jax.experimental.pallas import tpu as pltpu
from jax.experimental.pallas import tpu_sc as plsc

_LANES = 16


@functools.lru_cache(maxsize=None)
def _build_gather(B, N, K, D, KP):
    info = plsc.get_sparse_core_info()
    NC, NS = info.num_cores, info.num_subcores
    NW = NC * NS
    assert B % NW == 0
    bpw = B // NW
    rows_pw = bpw * K
    CH = 128
    assert rows_pw % CH == 0
    nchunk = rows_pw // CH
    GPS = 2
    assert nchunk % GPS == 0
    nslot = nchunk // GPS
    SR = GPS * CH
    NB = 4
    SKEW = 2

    mesh = plsc.VectorSubcoreMesh(core_axis_name="c", subcore_axis_name="s")

    @functools.partial(
        pl.kernel,
        mesh=mesh,
        compiler_params=pltpu.CompilerParams(
            use_tc_tiling_on_sc=False, needs_layout_passes=False
        ),
        out_type=jax.ShapeDtypeStruct((B * K, D), jnp.float32),
        scratch_types=[
            pltpu.VMEM((KP,), jnp.int32),
            pltpu.VMEM((nchunk, CH), jnp.int32),
            pltpu.VMEM((NB, SR, D), jnp.float32),
            pltpu.SemaphoreType.DMA((NB,)),
            pltpu.SemaphoreType.DMA((NB,)),
        ],
    )
    def gather_kernel(table_hbm, idx_hbm, out_hbm, idx_v, cidx_v, rows_v,
                      gsem, wsem):
        wid = lax.axis_index("s") * NC + lax.axis_index("c")
        pltpu.sync_copy(idx_hbm, idx_v)
        row0 = wid * rows_pw
        batch0 = wid * bpw

        def chunk_body(c, carry):
            p0 = c * CH
            for t in range(CH // _LANES):
                p = p0 + t * _LANES + lax.iota(jnp.int32, _LANES)
                b = lax.div(p, K)
                j = p - b * K
                src = (batch0 + b) * N + plsc.load_gather(idx_v, [j])
                cidx_v[c, pl.ds(t * _LANES, _LANES)] = src
            return carry

        lax.fori_loop(0, nchunk, chunk_body, 0)

        gathers = [None] * nslot
        writes = [None] * nslot
        for s in range(nslot + SKEW):
            if s < nslot:
                slot = s % NB
                if s >= NB:
                    writes[s - NB].wait()
                gathers[s] = [
                    pltpu.async_copy(
                        table_hbm.at[cidx_v.at[s * GPS + g]],
                        rows_v.at[slot, pl.ds(g * CH, CH)],
                        gsem.at[slot],
                    )
                    for g in range(GPS)
                ]
            w = s - SKEW
            if w >= 0:
                for h in gathers[w]:
                    h.wait()
                writes[w] = pltpu.async_copy(
                    rows_v.at[w % NB],
                    out_hbm.at[pl.ds(row0 + w * SR, SR)],
                    wsem.at[w % NB],
                )
        for w in range(max(0, nslot - NB), nslot):
            writes[w].wait()

    return gather_kernel


def kernel(tensor, indices):
    B, N, D = tensor.shape
    K = indices.shape[0]
    KP = (K + _LANES - 1) // _LANES * _LANES
    table = tensor.reshape(B * N, D)
    idx32 = jnp.pad(indices.astype(jnp.int32), (0, KP - K))
    out = _build_gather(B, N, K, D, KP)(table, idx32)
    return out.reshape(B, K, D)


# speedup vs baseline: 1.4111x; 1.3428x over previous
import functools

import jax
import jax.numpy as jnp
from jax import lax
from jax.experimental import pallas as pl
from jax.experimental.pallas import tpu as pltpu
from jax.experimental.pallas import tpu_sc as plsc

_LANES = 16


@functools.lru_cache(maxsize=None)
def _build_gather(B, N, K, D, KP):
    info = plsc.get_sparse_core_info()
    NC, NS = info.num_cores, info.num_subcores
    NW = NC * NS
    assert B % NW == 0
    bpw = B // NW
    NB = 6
    SKEW = 3

    mesh = plsc.VectorSubcoreMesh(core_axis_name="c", subcore_axis_name="s")

    @functools.partial(
        pl.kernel,
        mesh=mesh,
        compiler_params=pltpu.CompilerParams(
            use_tc_tiling_on_sc=True, needs_layout_passes=False
        ),
        out_type=jax.ShapeDtypeStruct((B, K, D), jnp.float32),
        scratch_types=[
            pltpu.VMEM((KP,), jnp.int32),
            pltpu.VMEM((NB, bpw, 1, D), jnp.float32),
            pltpu.SemaphoreType.DMA((NB,)),
            pltpu.SemaphoreType.DMA((NB,)),
        ],
    )
    def gather_kernel(tensor_hbm, idx_hbm, out_hbm, idx_v, bufs, gsem, wsem):
        wid = lax.axis_index("s") * NC + lax.axis_index("c")
        pltpu.sync_copy(idx_hbm, idx_v)
        b0 = wid * bpw
        lane = lax.iota(jnp.int32, _LANES)

        ij = []
        for c in range(KP // _LANES):
            vec = idx_v[pl.ds(c * _LANES, _LANES)]
            for l in range(_LANES):
                j = c * _LANES + l
                if j < K:
                    ij.append(jnp.sum(jnp.where(lane == l, vec, 0)))

        reads = [None] * K
        writes = [None] * K
        for s in range(K + SKEW):
            if s < K:
                slot = s % NB
                if s >= NB:
                    writes[s - NB].wait()
                reads[s] = pltpu.async_copy(
                    tensor_hbm.at[pl.ds(b0, bpw), pl.ds(ij[s], 1)],
                    bufs.at[slot],
                    gsem.at[slot],
                )
            w = s - SKEW
            if w >= 0:
                reads[w].wait()
                writes[w] = pltpu.async_copy(
                    bufs.at[w % NB],
                    out_hbm.at[pl.ds(b0, bpw), pl.ds(w, 1)],
                    wsem.at[w % NB],
                )
        for w in range(max(0, K - NB), K):
            writes[w].wait()

    return gather_kernel


def kernel(tensor, indices):
    B, N, D = tensor.shape
    K = indices.shape[0]
    KP = (K + _LANES - 1) // _LANES * _LANES
    idx32 = jnp.pad(indices.astype(jnp.int32), (0, KP - K))
    return _build_gather(B, N, K, D, KP)(tensor, idx32)
